# baseline (speedup 1.0000x reference)
import jax
import jax.numpy as jnp
from jax.experimental import pallas as pl
from jax.experimental.pallas import tpu as pltpu

_NSUB = 10
_FEAT = 48
_H = 32


def _labels_body(x_ref, wf_ref, ct_ref, out_ref):
    n = x_ref.shape[0]
    feats = jnp.dot(x_ref[...], wf_ref[...], preferred_element_type=jnp.float32)
    ct = ct_ref[...]
    cn = jnp.sum(ct * ct, axis=0, keepdims=True)
    d2 = cn - 2.0 * jnp.dot(feats, ct, preferred_element_type=jnp.float32)
    m = jnp.min(d2, axis=1, keepdims=True)
    iota = jax.lax.broadcasted_iota(jnp.int32, d2.shape, 1)
    cand = jnp.where(d2 == m, iota, _NSUB)
    lab = jnp.min(cand, axis=1, keepdims=True)

    onehot = (iota == lab).astype(jnp.float32)
    hist = jnp.sum(onehot, axis=0, keepdims=True)
    lt10 = (jax.lax.broadcasted_iota(jnp.int32, (_NSUB, _NSUB), 0) <
            jax.lax.broadcasted_iota(jnp.int32, (_NSUB, _NSUB), 1))
    csum = jnp.dot(hist, lt10.astype(jnp.float32),
                   preferred_element_type=jnp.float32)
    count_less = jnp.sum(onehot * csum, axis=1, keepdims=True)
    gtn = (jax.lax.broadcasted_iota(jnp.int32, (n, n), 1) <
           jax.lax.broadcasted_iota(jnp.int32, (n, n), 0)).astype(jnp.float32)
    cum_n = jnp.dot(gtn, onehot, preferred_element_type=jnp.float32)
    rank = jnp.sum(onehot * cum_n, axis=1, keepdims=True)
    pos = (count_less + rank).astype(jnp.int32)
    p = (jax.lax.broadcasted_iota(jnp.int32, (n, n), 1) == pos).astype(
        jnp.float32)
    iota_n = jax.lax.broadcasted_iota(jnp.int32, (1, n), 1).astype(jnp.float32)
    perm = jnp.dot(iota_n, p, preferred_element_type=jnp.float32)
    slab = jnp.dot(lab.astype(jnp.float32).reshape(1, n), p,
                   preferred_element_type=jnp.float32)
    out_ref[...] = jnp.concatenate([perm, slab], axis=0).astype(jnp.int32)


def _shift(x, off, axis):
    if off == 0:
        return x
    zshape = list(x.shape)
    zshape[axis] = 1
    z = jnp.zeros(zshape, x.dtype)
    if off == -1:
        body = jax.lax.slice_in_dim(x, 0, x.shape[axis] - 1, axis=axis)
        return jax.lax.concatenate([z, body], axis)
    body = jax.lax.slice_in_dim(x, 1, x.shape[axis], axis=axis)
    return jax.lax.concatenate([body, z], axis)


def _tree_sum(vals):
    while len(vals) > 1:
        nxt = [vals[i] + vals[i + 1] for i in range(0, len(vals) - 1, 2)]
        if len(vals) % 2:
            nxt.append(vals[-1])
        vals = nxt
    return vals[0]


def _conv3x3(x, w):
    H, W, Cin = x.shape
    Cout = w.shape[2]
    dots = []
    for ki in range(3):
        xr = _shift(x, ki - 1, 0)
        for kj in range(3):
            xc = _shift(xr, kj - 1, 1)
            dots.append(jnp.dot(
                xc.reshape(H * W, Cin), w[ki * 3 + kj],
                preferred_element_type=jnp.float32))
    return _tree_sum(dots).reshape(H, W, Cout)


def _expert_body(pm_ref, sl_ref, x_ref, hw_ref, r1_ref, r2_ref, u1_ref,
                 u2_ref, vt_ref, o_ref):
    x = x_ref[0]
    h = _conv3x3(x, hw_ref[0])
    r = _conv3x3(jnp.maximum(_conv3x3(h, r1_ref[0]), 0.0), r2_ref[0])
    h = h + r
    u1 = _conv3x3(h, u1_ref[0])

    sv = {}
    for ap in (0, 1):
        for bp in (0, 1):
            blk = u1[:, :, (2 * ap + bp) * _FEAT:(2 * ap + bp + 1) * _FEAT]
            for cy in ((0, 1) if ap == 0 else (-1, 0)):
                br = _shift(blk, cy, 0)
                for cx in ((0, 1) if bp == 0 else (-1, 0)):
                    sv[(ap, bp, cy, cx)] = _shift(br, cx, 1).reshape(
                        _H * _H, _FEAT)

    u2w = u2_ref[0]
    t = {}
    for a in (0, 1):
        for b in (0, 1):
            dots = []
            for oy in (-1, 0, 1):
                ap = (a + oy) % 2
                cy = (a + oy - ap) // 2
                for ox in (-1, 0, 1):
                    bp = (b + ox) % 2
                    cx = (b + ox - bp) // 2
                    tap = (oy + 1) * 3 + (ox + 1)
                    dots.append(jnp.dot(sv[(ap, bp, cy, cx)], u2w[tap],
                                        preferred_element_type=jnp.float32))
            t[(a, b)] = _tree_sum(dots)

    parts = []
    for a in (0, 1):
        for b in (0, 1):
            tab = t[(a, b)].reshape(_H, _H, 4 * _FEAT)
            dots = []
            for iy in (0, 1):
                sr = _shift(tab, iy - a, 0)
                for ix in (0, 1):
                    src = _shift(sr, ix - b, 1).reshape(_H * _H, 4 * _FEAT)
                    dots.append(jnp.dot(src, vt_ref[0, a * 2 + b, iy * 2 + ix],
                                        preferred_element_type=jnp.float32))
            parts.append(_tree_sum(dots))
    o_ref[0] = _tree_sum(parts).reshape(_H, _H, 48)


def _vt_index():
    idx = [[[[9] * 16 for _ in range(4)] for _ in range(4)] for _ in range(4)]
    for a in (0, 1):
        for b in (0, 1):
            for pr in range(4):
                for oy in (-1, 0, 1):
                    qr = pr + oy
                    cy = qr // 4
                    qm = qr % 4
                    if qm // 2 != a:
                        continue
                    alpha = qm % 2
                    iy = cy + a
                    for pc in range(4):
                        for ox in (-1, 0, 1):
                            qc = pc + ox
                            cx = qc // 4
                            qn = qc % 4
                            if qn // 2 != b:
                                continue
                            beta = qn % 2
                            ix = cx + b
                            idx[a * 2 + b][iy * 2 + ix][2 * alpha + beta][
                                4 * pr + pc] = (oy + 1) * 3 + (ox + 1)
    return jnp.asarray(idx, jnp.int32)


def _assemble_vt(twr):
    twe = jnp.concatenate(
        [twr, jnp.zeros((_NSUB, 1, _FEAT, 3), jnp.float32)], axis=1)
    vt = jnp.take(twe, _vt_index(), axis=1)
    vt = jnp.transpose(vt, (0, 1, 2, 3, 5, 4, 6))
    return vt.reshape(_NSUB, 4, 4, 4 * _FEAT, 48)


def kernel(inputs, W_feat, centroids, head_w, res1_w, res2_w, up1_w, up2_w,
           tail_w):
    n = inputs.shape[0]
    xflat = inputs.reshape(n, -1)
    route = pl.pallas_call(
        _labels_body,
        out_shape=jax.ShapeDtypeStruct((2, n), jnp.int32),
    )(xflat, W_feat, centroids.T)
    perm = route[0]
    slab = route[1]

    x = jnp.transpose(inputs, (0, 2, 3, 1))
    hw = head_w.reshape(_NSUB, 9, 3, _FEAT)
    r1 = res1_w.reshape(_NSUB, 9, _FEAT, _FEAT)
    r2 = res2_w.reshape(_NSUB, 9, _FEAT, _FEAT)
    u1 = up1_w.reshape(_NSUB, 9, _FEAT, _FEAT * 4)
    u2 = up2_w.reshape(_NSUB, 9, _FEAT, _FEAT * 4)
    vt = _assemble_vt(tail_w.reshape(_NSUB, 9, _FEAT, 3))

    def wspec(shape):
        return pl.BlockSpec(
            (1,) + shape,
            lambda i, pm, sl: (sl[i],) + (0,) * len(shape))

    out = pl.pallas_call(
        _expert_body,
        grid_spec=pltpu.PrefetchScalarGridSpec(
            num_scalar_prefetch=2,
            grid=(n,),
            in_specs=[
                pl.BlockSpec((1, _H, _H, 3), lambda i, pm, sl: (pm[i], 0, 0, 0)),
                wspec((9, 3, _FEAT)),
                wspec((9, _FEAT, _FEAT)),
                wspec((9, _FEAT, _FEAT)),
                wspec((9, _FEAT, _FEAT * 4)),
                wspec((9, _FEAT, _FEAT * 4)),
                wspec((4, 4, 4 * _FEAT, 48)),
            ],
            out_specs=pl.BlockSpec((1, _H, _H, 48),
                                   lambda i, pm, sl: (pm[i], 0, 0, 0)),
        ),
        out_shape=jax.ShapeDtypeStruct((n, _H, _H, 48), jnp.float32),
        compiler_params=pltpu.CompilerParams(
            dimension_semantics=("arbitrary",),
            vmem_limit_bytes=100 * 1024 * 1024,
        ),
    )(perm, slab, x, hw, r1, r2, u1, u2, vt)
    fine = out.reshape(n, _H, _H, 4, 4, 3)
    fine = jnp.transpose(fine, (0, 5, 1, 3, 2, 4))
    return fine.reshape(n, 3, 4 * _H, 4 * _H)


# speedup vs baseline: 1.1736x; 1.1736x over previous
import jax
import jax.numpy as jnp
from jax.experimental import pallas as pl
from jax.experimental.pallas import tpu as pltpu

_NSUB = 10
_FEAT = 48
_H = 32


def _labels_body(x_ref, wf_ref, ct_ref, out_ref):
    n = x_ref.shape[0]
    feats = jnp.dot(x_ref[...], wf_ref[...], preferred_element_type=jnp.float32)
    ct = ct_ref[...]
    cn = jnp.sum(ct * ct, axis=0, keepdims=True)
    d2 = cn - 2.0 * jnp.dot(feats, ct, preferred_element_type=jnp.float32)
    m = jnp.min(d2, axis=1, keepdims=True)
    iota = jax.lax.broadcasted_iota(jnp.int32, d2.shape, 1)
    cand = jnp.where(d2 == m, iota, _NSUB)
    lab = jnp.min(cand, axis=1, keepdims=True)

    onehot = (iota == lab).astype(jnp.float32)
    hist = jnp.sum(onehot, axis=0, keepdims=True)
    lt10 = (jax.lax.broadcasted_iota(jnp.int32, (_NSUB, _NSUB), 0) <
            jax.lax.broadcasted_iota(jnp.int32, (_NSUB, _NSUB), 1))
    csum = jnp.dot(hist, lt10.astype(jnp.float32),
                   preferred_element_type=jnp.float32)
    count_less = jnp.sum(onehot * csum, axis=1, keepdims=True)
    gtn = (jax.lax.broadcasted_iota(jnp.int32, (n, n), 1) <
           jax.lax.broadcasted_iota(jnp.int32, (n, n), 0)).astype(jnp.float32)
    cum_n = jnp.dot(gtn, onehot, preferred_element_type=jnp.float32)
    rank = jnp.sum(onehot * cum_n, axis=1, keepdims=True)
    pos = (count_less + rank).astype(jnp.int32)
    p = (jax.lax.broadcasted_iota(jnp.int32, (n, n), 1) == pos).astype(
        jnp.float32)
    iota_n = jax.lax.broadcasted_iota(jnp.int32, (1, n), 1).astype(jnp.float32)
    perm = jnp.dot(iota_n, p, preferred_element_type=jnp.float32)
    slab = jnp.dot(lab.astype(jnp.float32).reshape(1, n), p,
                   preferred_element_type=jnp.float32)
    out_ref[...] = jnp.concatenate([perm, slab], axis=0).astype(jnp.int32)


def _shift(x, off, axis):
    if off == 0:
        return x
    zshape = list(x.shape)
    zshape[axis] = 1
    z = jnp.zeros(zshape, x.dtype)
    if off == -1:
        body = jax.lax.slice_in_dim(x, 0, x.shape[axis] - 1, axis=axis)
        return jax.lax.concatenate([z, body], axis)
    body = jax.lax.slice_in_dim(x, 1, x.shape[axis], axis=axis)
    return jax.lax.concatenate([body, z], axis)


def _tree_sum(vals):
    while len(vals) > 1:
        nxt = [vals[i] + vals[i + 1] for i in range(0, len(vals) - 1, 2)]
        if len(vals) % 2:
            nxt.append(vals[-1])
        vals = nxt
    return vals[0]


def _conv3x3(x, w):
    H, W, Cin = x.shape
    Cout = w.shape[2]
    dots = []
    for ki in range(3):
        xr = _shift(x, ki - 1, 0)
        for kj in range(3):
            xc = _shift(xr, kj - 1, 1)
            dots.append(jnp.dot(
                xc.reshape(H * W, Cin), w[ki * 3 + kj],
                preferred_element_type=jnp.float32))
    return _tree_sum(dots).reshape(H, W, Cout)


def _expert_body(pm_ref, sl_ref, x_ref, hw_ref, r1_ref, r2_ref, u1_ref,
                 v2_ref, vt_ref, o_ref):
    x = x_ref[0]
    h = _conv3x3(x, hw_ref[0])
    r = _conv3x3(jnp.maximum(_conv3x3(h, r1_ref[0]), 0.0), r2_ref[0])
    h = h + r
    u1 = _conv3x3(h, u1_ref[0])

    s = {}
    for cy in (-1, 0, 1):
        ur = _shift(u1, cy, 0)
        for cx in (-1, 0, 1):
            s[(cy, cx)] = _shift(ur, cx, 1).reshape(_H * _H, 4 * _FEAT)

    t = {}
    for a in (0, 1):
        for b in (0, 1):
            dots = []
            for iy in (0, 1):
                for ix in (0, 1):
                    v = v2_ref[0, a * 2 + b, iy * 2 + ix]
                    dots.append(jnp.dot(s[(iy - 1 + a, ix - 1 + b)], v,
                                        preferred_element_type=jnp.float32))
            t[(a, b)] = _tree_sum(dots)

    parts = []
    for a in (0, 1):
        for b in (0, 1):
            tab = t[(a, b)].reshape(_H, _H, 4 * _FEAT)
            dots = []
            for iy in (0, 1):
                sr = _shift(tab, iy - a, 0)
                for ix in (0, 1):
                    src = _shift(sr, ix - b, 1).reshape(_H * _H, 4 * _FEAT)
                    dots.append(jnp.dot(src, vt_ref[0, a * 2 + b, iy * 2 + ix],
                                        preferred_element_type=jnp.float32))
            parts.append(_tree_sum(dots))
    o_ref[0] = _tree_sum(parts).reshape(_H, _H, 48)


def _v2_index():
    idx = [[[9] * 4 for _ in range(4)] for _ in range(4)]
    for a in (0, 1):
        for b in (0, 1):
            for oy in (-1, 0, 1):
                ap = (a + oy) % 2
                cy = (a + oy - ap) // 2
                iy = cy + 1 - a
                for ox in (-1, 0, 1):
                    bp = (b + ox) % 2
                    cx = (b + ox - bp) // 2
                    ix = cx + 1 - b
                    idx[a * 2 + b][iy * 2 + ix][2 * ap + bp] = \
                        (oy + 1) * 3 + (ox + 1)
    return jnp.asarray(idx, jnp.int32)


def _assemble_v2(u2r):
    u2e = jnp.concatenate(
        [u2r, jnp.zeros((_NSUB, 1, _FEAT, 4 * _FEAT), jnp.float32)], axis=1)
    v2 = jnp.take(u2e, _v2_index(), axis=1)
    return v2.reshape(_NSUB, 4, 4, 4 * _FEAT, 4 * _FEAT)


def _vt_index():
    idx = [[[[9] * 16 for _ in range(4)] for _ in range(4)] for _ in range(4)]
    for a in (0, 1):
        for b in (0, 1):
            for pr in range(4):
                for oy in (-1, 0, 1):
                    qr = pr + oy
                    cy = qr // 4
                    qm = qr % 4
                    if qm // 2 != a:
                        continue
                    alpha = qm % 2
                    iy = cy + a
                    for pc in range(4):
                        for ox in (-1, 0, 1):
                            qc = pc + ox
                            cx = qc // 4
                            qn = qc % 4
                            if qn // 2 != b:
                                continue
                            beta = qn % 2
                            ix = cx + b
                            idx[a * 2 + b][iy * 2 + ix][2 * alpha + beta][
                                4 * pr + pc] = (oy + 1) * 3 + (ox + 1)
    return jnp.asarray(idx, jnp.int32)


def _assemble_vt(twr):
    twe = jnp.concatenate(
        [twr, jnp.zeros((_NSUB, 1, _FEAT, 3), jnp.float32)], axis=1)
    vt = jnp.take(twe, _vt_index(), axis=1)
    vt = jnp.transpose(vt, (0, 1, 2, 3, 5, 4, 6))
    return vt.reshape(_NSUB, 4, 4, 4 * _FEAT, 48)


def kernel(inputs, W_feat, centroids, head_w, res1_w, res2_w, up1_w, up2_w,
           tail_w):
    n = inputs.shape[0]
    xflat = inputs.reshape(n, -1)
    route = pl.pallas_call(
        _labels_body,
        out_shape=jax.ShapeDtypeStruct((2, n), jnp.int32),
    )(xflat, W_feat, centroids.T)
    perm = route[0]
    slab = route[1]

    x = jnp.transpose(inputs, (0, 2, 3, 1))
    hw = head_w.reshape(_NSUB, 9, 3, _FEAT)
    r1 = res1_w.reshape(_NSUB, 9, _FEAT, _FEAT)
    r2 = res2_w.reshape(_NSUB, 9, _FEAT, _FEAT)
    u1 = up1_w.reshape(_NSUB, 9, _FEAT, _FEAT * 4)
    v2 = _assemble_v2(up2_w.reshape(_NSUB, 9, _FEAT, _FEAT * 4))
    vt = _assemble_vt(tail_w.reshape(_NSUB, 9, _FEAT, 3))

    def wspec(shape):
        return pl.BlockSpec(
            (1,) + shape,
            lambda i, pm, sl: (sl[i],) + (0,) * len(shape))

    out = pl.pallas_call(
        _expert_body,
        grid_spec=pltpu.PrefetchScalarGridSpec(
            num_scalar_prefetch=2,
            grid=(n,),
            in_specs=[
                pl.BlockSpec((1, _H, _H, 3), lambda i, pm, sl: (pm[i], 0, 0, 0)),
                wspec((9, 3, _FEAT)),
                wspec((9, _FEAT, _FEAT)),
                wspec((9, _FEAT, _FEAT)),
                wspec((9, _FEAT, _FEAT * 4)),
                wspec((4, 4, 4 * _FEAT, 4 * _FEAT)),
                wspec((4, 4, 4 * _FEAT, 48)),
            ],
            out_specs=pl.BlockSpec((1, _H, _H, 48),
                                   lambda i, pm, sl: (pm[i], 0, 0, 0)),
        ),
        out_shape=jax.ShapeDtypeStruct((n, _H, _H, 48), jnp.float32),
        compiler_params=pltpu.CompilerParams(
            dimension_semantics=("arbitrary",),
            vmem_limit_bytes=100 * 1024 * 1024,
        ),
    )(perm, slab, x, hw, r1, r2, u1, v2, vt)
    fine = out.reshape(n, _H, _H, 4, 4, 3)
    fine = jnp.transpose(fine, (0, 5, 1, 3, 2, 4))
    return fine.reshape(n, 3, 4 * _H, 4 * _H)


# speedup vs baseline: 1.2208x; 1.0403x over previous
import jax
import jax.numpy as jnp
from jax.experimental import pallas as pl
from jax.experimental.pallas import tpu as pltpu

_NSUB = 10
_FEAT = 48
_H = 32


def _labels_body(x_ref, wf_ref, ct_ref, out_ref):
    n = x_ref.shape[0]
    feats = jnp.dot(x_ref[...], wf_ref[...], preferred_element_type=jnp.float32)
    ct = ct_ref[...]
    cn = jnp.sum(ct * ct, axis=0, keepdims=True)
    d2 = cn - 2.0 * jnp.dot(feats, ct, preferred_element_type=jnp.float32)
    m = jnp.min(d2, axis=1, keepdims=True)
    iota = jax.lax.broadcasted_iota(jnp.int32, d2.shape, 1)
    cand = jnp.where(d2 == m, iota, _NSUB)
    lab = jnp.min(cand, axis=1, keepdims=True)

    onehot = (iota == lab).astype(jnp.float32)
    hist = jnp.sum(onehot, axis=0, keepdims=True)
    lt10 = (jax.lax.broadcasted_iota(jnp.int32, (_NSUB, _NSUB), 0) <
            jax.lax.broadcasted_iota(jnp.int32, (_NSUB, _NSUB), 1))
    csum = jnp.dot(hist, lt10.astype(jnp.float32),
                   preferred_element_type=jnp.float32)
    count_less = jnp.sum(onehot * csum, axis=1, keepdims=True)
    gtn = (jax.lax.broadcasted_iota(jnp.int32, (n, n), 1) <
           jax.lax.broadcasted_iota(jnp.int32, (n, n), 0)).astype(jnp.float32)
    cum_n = jnp.dot(gtn, onehot, preferred_element_type=jnp.float32)
    rank = jnp.sum(onehot * cum_n, axis=1, keepdims=True)
    pos = (count_less + rank).astype(jnp.int32)
    p = (jax.lax.broadcasted_iota(jnp.int32, (n, n), 1) == pos).astype(
        jnp.float32)
    iota_n = jax.lax.broadcasted_iota(jnp.int32, (1, n), 1).astype(jnp.float32)
    perm = jnp.dot(iota_n, p, preferred_element_type=jnp.float32)
    slab = jnp.dot(lab.astype(jnp.float32).reshape(1, n), p,
                   preferred_element_type=jnp.float32)
    out_ref[...] = jnp.concatenate([perm, slab], axis=0).astype(jnp.int32)


def _shift(x, off, axis):
    if off == 0:
        return x
    zshape = list(x.shape)
    zshape[axis] = 1
    z = jnp.zeros(zshape, x.dtype)
    if off == -1:
        body = jax.lax.slice_in_dim(x, 0, x.shape[axis] - 1, axis=axis)
        return jax.lax.concatenate([z, body], axis)
    body = jax.lax.slice_in_dim(x, 1, x.shape[axis], axis=axis)
    return jax.lax.concatenate([body, z], axis)


def _conv3x3(x, w):
    H, W, Cin = x.shape
    Cout = w.shape[2]
    acc = jnp.zeros((H * W, Cout), jnp.float32)
    for ki in range(3):
        xr = _shift(x, ki - 1, 0)
        for kj in range(3):
            xc = _shift(xr, kj - 1, 1)
            acc = acc + jnp.dot(
                xc.reshape(H * W, Cin), w[ki * 3 + kj],
                preferred_element_type=jnp.float32)
    return acc.reshape(H, W, Cout)


def _expert_body(pm_ref, sl_ref, x_ref, hw_ref, r1_ref, r2_ref, u1_ref,
                 v2_ref, vt_ref, o_ref):
    x = x_ref[0]
    h = _conv3x3(x, hw_ref[0])
    r = _conv3x3(jnp.maximum(_conv3x3(h, r1_ref[0]), 0.0), r2_ref[0])
    h = h + r
    u1 = _conv3x3(h, u1_ref[0])

    s = {}
    for cy in (-1, 0, 1):
        ur = _shift(u1, cy, 0)
        for cx in (-1, 0, 1):
            s[(cy, cx)] = _shift(ur, cx, 1).reshape(_H * _H, 4 * _FEAT)

    t = {}
    for a in (0, 1):
        for b in (0, 1):
            acc = jnp.zeros((_H * _H, 4 * _FEAT), jnp.float32)
            for iy in (0, 1):
                for ix in (0, 1):
                    v = v2_ref[0, a * 2 + b, iy * 2 + ix]
                    acc = acc + jnp.dot(s[(iy - 1 + a, ix - 1 + b)], v,
                                        preferred_element_type=jnp.float32)
            t[(a, b)] = acc

    out = jnp.zeros((_H * _H, 48), jnp.float32)
    for a in (0, 1):
        for b in (0, 1):
            tab = t[(a, b)].reshape(_H, _H, 4 * _FEAT)
            for iy in (0, 1):
                sr = _shift(tab, iy - a, 0)
                for ix in (0, 1):
                    src = _shift(sr, ix - b, 1).reshape(_H * _H, 4 * _FEAT)
                    out = out + jnp.dot(src, vt_ref[0, a * 2 + b, iy * 2 + ix],
                                        preferred_element_type=jnp.float32)
    o_ref[0] = out.reshape(_H, _H, 48)


def _v2_index():
    idx = [[[9] * 4 for _ in range(4)] for _ in range(4)]
    for a in (0, 1):
        for b in (0, 1):
            for oy in (-1, 0, 1):
                ap = (a + oy) % 2
                cy = (a + oy - ap) // 2
                iy = cy + 1 - a
                for ox in (-1, 0, 1):
                    bp = (b + ox) % 2
                    cx = (b + ox - bp) // 2
                    ix = cx + 1 - b
                    idx[a * 2 + b][iy * 2 + ix][2 * ap + bp] = \
                        (oy + 1) * 3 + (ox + 1)
    return idx


def _assemble_v2(u2r):
    idx = _v2_index()
    zero = jnp.zeros((_NSUB, _FEAT, 4 * _FEAT), jnp.float32)
    mats = []
    for ab in range(4):
        for s in range(4):
            ks = [u2r[:, int(idx[ab][s][kb])] if int(idx[ab][s][kb]) < 9
                  else zero for kb in range(4)]
            mats.append(jnp.concatenate(ks, axis=1))
    v2 = jnp.stack(mats, axis=1)
    return v2.reshape(_NSUB, 4, 4, 4 * _FEAT, 4 * _FEAT)


def _vt_index():
    idx = [[[[9] * 16 for _ in range(4)] for _ in range(4)] for _ in range(4)]
    for a in (0, 1):
        for b in (0, 1):
            for pr in range(4):
                for oy in (-1, 0, 1):
                    qr = pr + oy
                    cy = qr // 4
                    qm = qr % 4
                    if qm // 2 != a:
                        continue
                    alpha = qm % 2
                    iy = cy + a
                    for pc in range(4):
                        for ox in (-1, 0, 1):
                            qc = pc + ox
                            cx = qc // 4
                            qn = qc % 4
                            if qn // 2 != b:
                                continue
                            beta = qn % 2
                            ix = cx + b
                            idx[a * 2 + b][iy * 2 + ix][2 * alpha + beta][
                                4 * pr + pc] = (oy + 1) * 3 + (ox + 1)
    return idx


def _assemble_vt(twr):
    idx = _vt_index()
    zero = jnp.zeros((_NSUB, _FEAT, 3), jnp.float32)
    mats = []
    for ab in range(4):
        for s in range(4):
            ks = []
            for kb in range(4):
                ns = [twr[:, int(idx[ab][s][kb][p])]
                      if int(idx[ab][s][kb][p]) < 9 else zero
                      for p in range(16)]
                ks.append(jnp.concatenate(ns, axis=2))
            mats.append(jnp.concatenate(ks, axis=1))
    vt = jnp.stack(mats, axis=1)
    return vt.reshape(_NSUB, 4, 4, 4 * _FEAT, 48)


def kernel(inputs, W_feat, centroids, head_w, res1_w, res2_w, up1_w, up2_w,
           tail_w):
    n = inputs.shape[0]
    xflat = inputs.reshape(n, -1)
    route = pl.pallas_call(
        _labels_body,
        out_shape=jax.ShapeDtypeStruct((2, n), jnp.int32),
    )(xflat, W_feat, centroids.T)
    perm = route[0]
    slab = route[1]

    x = jnp.transpose(inputs, (0, 2, 3, 1))
    hw = head_w.reshape(_NSUB, 9, 3, _FEAT)
    r1 = res1_w.reshape(_NSUB, 9, _FEAT, _FEAT)
    r2 = res2_w.reshape(_NSUB, 9, _FEAT, _FEAT)
    u1 = up1_w.reshape(_NSUB, 9, _FEAT, _FEAT * 4)
    v2 = _assemble_v2(up2_w.reshape(_NSUB, 9, _FEAT, _FEAT * 4))
    vt = _assemble_vt(tail_w.reshape(_NSUB, 9, _FEAT, 3))

    def wspec(shape):
        return pl.BlockSpec(
            (1,) + shape,
            lambda i, pm, sl: (sl[i],) + (0,) * len(shape))

    out = pl.pallas_call(
        _expert_body,
        grid_spec=pltpu.PrefetchScalarGridSpec(
            num_scalar_prefetch=2,
            grid=(n,),
            in_specs=[
                pl.BlockSpec((1, _H, _H, 3), lambda i, pm, sl: (pm[i], 0, 0, 0)),
                wspec((9, 3, _FEAT)),
                wspec((9, _FEAT, _FEAT)),
                wspec((9, _FEAT, _FEAT)),
                wspec((9, _FEAT, _FEAT * 4)),
                wspec((4, 4, 4 * _FEAT, 4 * _FEAT)),
                wspec((4, 4, 4 * _FEAT, 48)),
            ],
            out_specs=pl.BlockSpec((1, _H, _H, 48),
                                   lambda i, pm, sl: (pm[i], 0, 0, 0)),
        ),
        out_shape=jax.ShapeDtypeStruct((n, _H, _H, 48), jnp.float32),
        compiler_params=pltpu.CompilerParams(
            dimension_semantics=("arbitrary",),
            vmem_limit_bytes=100 * 1024 * 1024,
        ),
    )(perm, slab, x, hw, r1, r2, u1, v2, vt)
    fine = out.reshape(n, _H, _H, 4, 4, 3)
    fine = jnp.transpose(fine, (0, 5, 1, 3, 2, 4))
    return fine.reshape(n, 3, 4 * _H, 4 * _H)


# speedup vs baseline: 1.4723x; 1.2060x over previous
import jax
import jax.numpy as jnp
from jax.experimental import pallas as pl
from jax.experimental.pallas import tpu as pltpu

_NSUB = 10
_FEAT = 48
_H = 32


def _labels_body(x_ref, wf_ref, ct_ref, out_ref):
    n = x_ref.shape[0]
    feats = jnp.dot(x_ref[...], wf_ref[...], preferred_element_type=jnp.float32)
    ct = ct_ref[...]
    cn = jnp.sum(ct * ct, axis=0, keepdims=True)
    d2 = cn - 2.0 * jnp.dot(feats, ct, preferred_element_type=jnp.float32)
    m = jnp.min(d2, axis=1, keepdims=True)
    iota = jax.lax.broadcasted_iota(jnp.int32, d2.shape, 1)
    cand = jnp.where(d2 == m, iota, _NSUB)
    lab = jnp.min(cand, axis=1, keepdims=True)

    onehot = (iota == lab).astype(jnp.float32)
    hist = jnp.sum(onehot, axis=0, keepdims=True)
    lt10 = (jax.lax.broadcasted_iota(jnp.int32, (_NSUB, _NSUB), 0) <
            jax.lax.broadcasted_iota(jnp.int32, (_NSUB, _NSUB), 1))
    csum = jnp.dot(hist, lt10.astype(jnp.float32),
                   preferred_element_type=jnp.float32)
    count_less = jnp.sum(onehot * csum, axis=1, keepdims=True)
    gtn = (jax.lax.broadcasted_iota(jnp.int32, (n, n), 1) <
           jax.lax.broadcasted_iota(jnp.int32, (n, n), 0)).astype(jnp.float32)
    cum_n = jnp.dot(gtn, onehot, preferred_element_type=jnp.float32)
    rank = jnp.sum(onehot * cum_n, axis=1, keepdims=True)
    pos = (count_less + rank).astype(jnp.int32)
    p = (jax.lax.broadcasted_iota(jnp.int32, (n, n), 1) == pos).astype(
        jnp.float32)
    iota_n = jax.lax.broadcasted_iota(jnp.int32, (1, n), 1).astype(jnp.float32)
    perm = jnp.dot(iota_n, p, preferred_element_type=jnp.float32)
    slab = jnp.dot(lab.astype(jnp.float32).reshape(1, n), p,
                   preferred_element_type=jnp.float32)
    out_ref[...] = jnp.concatenate([perm, slab], axis=0).astype(jnp.int32)


def _shift(x, off, axis):
    if off == 0:
        return x
    zshape = list(x.shape)
    zshape[axis] = 1
    z = jnp.zeros(zshape, x.dtype)
    if off == -1:
        body = jax.lax.slice_in_dim(x, 0, x.shape[axis] - 1, axis=axis)
        return jax.lax.concatenate([z, body], axis)
    body = jax.lax.slice_in_dim(x, 1, x.shape[axis], axis=axis)
    return jax.lax.concatenate([body, z], axis)


def _conv3x3(x, w):
    H, W, Cin = x.shape
    Cout = w.shape[2]
    acc = jnp.zeros((H * W, Cout), jnp.float32)
    for ki in range(3):
        xr = _shift(x, ki - 1, 0)
        for kj in range(3):
            xc = _shift(xr, kj - 1, 1)
            acc = acc + jnp.dot(
                xc.reshape(H * W, Cin), w[ki * 3 + kj],
                preferred_element_type=jnp.float32)
    return acc.reshape(H, W, Cout)


def _conv3x3_im2col(x, w):
    H, W, Cin = x.shape
    Cout = w.shape[2]
    cols = []
    for ki in range(3):
        xr = _shift(x, ki - 1, 0)
        for kj in range(3):
            cols.append(_shift(xr, kj - 1, 1).reshape(H * W, Cin))
    patches = jnp.concatenate(cols, axis=1)
    out = jnp.dot(patches, w.reshape(9 * Cin, Cout),
                  preferred_element_type=jnp.float32)
    return out.reshape(H, W, Cout)


def _expert_body(pm_ref, sl_ref, x_ref, hw_ref, r1_ref, r2_ref, u1_ref,
                 v2_ref, vt_ref, o_ref):
    x = x_ref[0]
    h = _conv3x3(x, hw_ref[0])
    r = _conv3x3_im2col(jnp.maximum(_conv3x3_im2col(h, r1_ref[0]), 0.0),
                        r2_ref[0])
    h = h + r
    u1 = _conv3x3_im2col(h, u1_ref[0])

    s = {}
    for cy in (-1, 0, 1):
        ur = _shift(u1, cy, 0)
        for cx in (-1, 0, 1):
            s[(cy, cx)] = _shift(ur, cx, 1).reshape(_H * _H, 4 * _FEAT)

    t = {}
    for a in (0, 1):
        for b in (0, 1):
            acc = jnp.zeros((_H * _H, 4 * _FEAT), jnp.float32)
            for iy in (0, 1):
                for ix in (0, 1):
                    v = v2_ref[0, a * 2 + b, iy * 2 + ix]
                    acc = acc + jnp.dot(s[(iy - 1 + a, ix - 1 + b)], v,
                                        preferred_element_type=jnp.float32)
            t[(a, b)] = acc

    out = jnp.zeros((_H * _H, 48), jnp.float32)
    for a in (0, 1):
        for b in (0, 1):
            tab = t[(a, b)].reshape(_H, _H, 4 * _FEAT)
            for iy in (0, 1):
                sr = _shift(tab, iy - a, 0)
                for ix in (0, 1):
                    src = _shift(sr, ix - b, 1).reshape(_H * _H, 4 * _FEAT)
                    out = out + jnp.dot(src, vt_ref[0, a * 2 + b, iy * 2 + ix],
                                        preferred_element_type=jnp.float32)
    o_ref[0] = out.reshape(_H, _H, 48)


def _v2_index():
    idx = [[[9] * 4 for _ in range(4)] for _ in range(4)]
    for a in (0, 1):
        for b in (0, 1):
            for oy in (-1, 0, 1):
                ap = (a + oy) % 2
                cy = (a + oy - ap) // 2
                iy = cy + 1 - a
                for ox in (-1, 0, 1):
                    bp = (b + ox) % 2
                    cx = (b + ox - bp) // 2
                    ix = cx + 1 - b
                    idx[a * 2 + b][iy * 2 + ix][2 * ap + bp] = \
                        (oy + 1) * 3 + (ox + 1)
    return idx


def _assemble_v2(u2r):
    idx = _v2_index()
    zero = jnp.zeros((_NSUB, _FEAT, 4 * _FEAT), jnp.float32)
    mats = []
    for ab in range(4):
        for s in range(4):
            ks = [u2r[:, int(idx[ab][s][kb])] if int(idx[ab][s][kb]) < 9
                  else zero for kb in range(4)]
            mats.append(jnp.concatenate(ks, axis=1))
    v2 = jnp.stack(mats, axis=1)
    return v2.reshape(_NSUB, 4, 4, 4 * _FEAT, 4 * _FEAT)


def _vt_index():
    idx = [[[[9] * 16 for _ in range(4)] for _ in range(4)] for _ in range(4)]
    for a in (0, 1):
        for b in (0, 1):
            for pr in range(4):
                for oy in (-1, 0, 1):
                    qr = pr + oy
                    cy = qr // 4
                    qm = qr % 4
                    if qm // 2 != a:
                        continue
                    alpha = qm % 2
                    iy = cy + a
                    for pc in range(4):
                        for ox in (-1, 0, 1):
                            qc = pc + ox
                            cx = qc // 4
                            qn = qc % 4
                            if qn // 2 != b:
                                continue
                            beta = qn % 2
                            ix = cx + b
                            idx[a * 2 + b][iy * 2 + ix][2 * alpha + beta][
                                4 * pr + pc] = (oy + 1) * 3 + (ox + 1)
    return idx


def _assemble_vt(twr):
    idx = _vt_index()
    zero = jnp.zeros((_NSUB, _FEAT, 3), jnp.float32)
    mats = []
    for ab in range(4):
        for s in range(4):
            ks = []
            for kb in range(4):
                ns = [twr[:, int(idx[ab][s][kb][p])]
                      if int(idx[ab][s][kb][p]) < 9 else zero
                      for p in range(16)]
                ks.append(jnp.concatenate(ns, axis=2))
            mats.append(jnp.concatenate(ks, axis=1))
    vt = jnp.stack(mats, axis=1)
    return vt.reshape(_NSUB, 4, 4, 4 * _FEAT, 48)


def kernel(inputs, W_feat, centroids, head_w, res1_w, res2_w, up1_w, up2_w,
           tail_w):
    n = inputs.shape[0]
    xflat = inputs.reshape(n, -1)
    route = pl.pallas_call(
        _labels_body,
        out_shape=jax.ShapeDtypeStruct((2, n), jnp.int32),
    )(xflat, W_feat, centroids.T)
    perm = route[0]
    slab = route[1]

    x = jnp.transpose(inputs, (0, 2, 3, 1))
    hw = head_w.reshape(_NSUB, 9, 3, _FEAT)
    r1 = res1_w.reshape(_NSUB, 9, _FEAT, _FEAT)
    r2 = res2_w.reshape(_NSUB, 9, _FEAT, _FEAT)
    u1 = up1_w.reshape(_NSUB, 9, _FEAT, _FEAT * 4)
    v2 = _assemble_v2(up2_w.reshape(_NSUB, 9, _FEAT, _FEAT * 4))
    vt = _assemble_vt(tail_w.reshape(_NSUB, 9, _FEAT, 3))

    def wspec(shape):
        return pl.BlockSpec(
            (1,) + shape,
            lambda i, pm, sl: (sl[i],) + (0,) * len(shape))

    out = pl.pallas_call(
        _expert_body,
        grid_spec=pltpu.PrefetchScalarGridSpec(
            num_scalar_prefetch=2,
            grid=(n,),
            in_specs=[
                pl.BlockSpec((1, _H, _H, 3), lambda i, pm, sl: (pm[i], 0, 0, 0)),
                wspec((9, 3, _FEAT)),
                wspec((9, _FEAT, _FEAT)),
                wspec((9, _FEAT, _FEAT)),
                wspec((9, _FEAT, _FEAT * 4)),
                wspec((4, 4, 4 * _FEAT, 4 * _FEAT)),
                wspec((4, 4, 4 * _FEAT, 48)),
            ],
            out_specs=pl.BlockSpec((1, _H, _H, 48),
                                   lambda i, pm, sl: (pm[i], 0, 0, 0)),
        ),
        out_shape=jax.ShapeDtypeStruct((n, _H, _H, 48), jnp.float32),
        compiler_params=pltpu.CompilerParams(
            dimension_semantics=("arbitrary",),
            vmem_limit_bytes=100 * 1024 * 1024,
        ),
    )(perm, slab, x, hw, r1, r2, u1, v2, vt)
    fine = out.reshape(n, _H, _H, 4, 4, 3)
    fine = jnp.transpose(fine, (0, 5, 1, 3, 2, 4))
    return fine.reshape(n, 3, 4 * _H, 4 * _H)


# speedup vs baseline: 1.5287x; 1.0384x over previous
import jax
import jax.numpy as jnp
from jax.experimental import pallas as pl
from jax.experimental.pallas import tpu as pltpu

_NSUB = 10
_FEAT = 48
_H = 32


def _labels_body(x_ref, wf_ref, ct_ref, out_ref):
    n = x_ref.shape[0]
    feats = jnp.dot(x_ref[...], wf_ref[...], preferred_element_type=jnp.float32)
    ct = ct_ref[...]
    cn = jnp.sum(ct * ct, axis=0, keepdims=True)
    d2 = cn - 2.0 * jnp.dot(feats, ct, preferred_element_type=jnp.float32)
    m = jnp.min(d2, axis=1, keepdims=True)
    iota = jax.lax.broadcasted_iota(jnp.int32, d2.shape, 1)
    cand = jnp.where(d2 == m, iota, _NSUB)
    lab = jnp.min(cand, axis=1, keepdims=True)

    onehot = (iota == lab).astype(jnp.float32)
    hist = jnp.sum(onehot, axis=0, keepdims=True)
    lt10 = (jax.lax.broadcasted_iota(jnp.int32, (_NSUB, _NSUB), 0) <
            jax.lax.broadcasted_iota(jnp.int32, (_NSUB, _NSUB), 1))
    csum = jnp.dot(hist, lt10.astype(jnp.float32),
                   preferred_element_type=jnp.float32)
    count_less = jnp.sum(onehot * csum, axis=1, keepdims=True)
    gtn = (jax.lax.broadcasted_iota(jnp.int32, (n, n), 1) <
           jax.lax.broadcasted_iota(jnp.int32, (n, n), 0)).astype(jnp.float32)
    cum_n = jnp.dot(gtn, onehot, preferred_element_type=jnp.float32)
    rank = jnp.sum(onehot * cum_n, axis=1, keepdims=True)
    pos = (count_less + rank).astype(jnp.int32)
    p = (jax.lax.broadcasted_iota(jnp.int32, (n, n), 1) == pos).astype(
        jnp.float32)
    iota_n = jax.lax.broadcasted_iota(jnp.int32, (1, n), 1).astype(jnp.float32)
    perm = jnp.dot(iota_n, p, preferred_element_type=jnp.float32)
    slab = jnp.dot(lab.astype(jnp.float32).reshape(1, n), p,
                   preferred_element_type=jnp.float32)
    out_ref[...] = jnp.concatenate([perm, slab], axis=0).astype(jnp.int32)


def _shift(x, off, axis):
    if off == 0:
        return x
    zshape = list(x.shape)
    zshape[axis] = 1
    z = jnp.zeros(zshape, x.dtype)
    if off == -1:
        body = jax.lax.slice_in_dim(x, 0, x.shape[axis] - 1, axis=axis)
        return jax.lax.concatenate([z, body], axis)
    body = jax.lax.slice_in_dim(x, 1, x.shape[axis], axis=axis)
    return jax.lax.concatenate([body, z], axis)


def _conv3x3(x, w):
    H, W, Cin = x.shape
    Cout = w.shape[2]
    acc = jnp.zeros((H * W, Cout), jnp.float32)
    for ki in range(3):
        xr = _shift(x, ki - 1, 0)
        for kj in range(3):
            xc = _shift(xr, kj - 1, 1)
            acc = acc + jnp.dot(
                xc.reshape(H * W, Cin), w[ki * 3 + kj],
                preferred_element_type=jnp.float32)
    return acc.reshape(H, W, Cout)


def _conv3x3_im2col(x, w):
    H, W, Cin = x.shape
    Cout = w.shape[2]
    cols = []
    for ki in range(3):
        xr = _shift(x, ki - 1, 0)
        for kj in range(3):
            cols.append(_shift(xr, kj - 1, 1).reshape(H * W, Cin))
    patches = jnp.concatenate(cols, axis=1)
    out = jnp.dot(patches, w.reshape(9 * Cin, Cout),
                  preferred_element_type=jnp.float32)
    return out.reshape(H, W, Cout)


def _expert_body(pm_ref, sl_ref, x_ref, hw_ref, r1_ref, r2_ref, u1_ref,
                 v2_ref, vt_ref, o_ref):
    x = x_ref[0]
    h = _conv3x3(x, hw_ref[0])
    r = _conv3x3_im2col(jnp.maximum(_conv3x3_im2col(h, r1_ref[0]), 0.0),
                        r2_ref[0])
    h = h + r
    u1 = _conv3x3_im2col(h, u1_ref[0])

    s = {}
    for cy in (-1, 0, 1):
        ur = _shift(u1, cy, 0)
        for cx in (-1, 0, 1):
            s[(cy, cx)] = _shift(ur, cx, 1).reshape(_H * _H, 4 * _FEAT)

    t = {}
    for a in (0, 1):
        for b in (0, 1):
            patches = jnp.concatenate(
                [s[(iy - 1 + a, ix - 1 + b)]
                 for iy in (0, 1) for ix in (0, 1)], axis=1)
            t[(a, b)] = jnp.dot(
                patches, v2_ref[0, a * 2 + b].reshape(16 * _FEAT, 4 * _FEAT),
                preferred_element_type=jnp.float32)

    out = jnp.zeros((_H * _H, 48), jnp.float32)
    for a in (0, 1):
        for b in (0, 1):
            tab = t[(a, b)].reshape(_H, _H, 4 * _FEAT)
            cols = []
            for iy in (0, 1):
                sr = _shift(tab, iy - a, 0)
                for ix in (0, 1):
                    cols.append(_shift(sr, ix - b, 1).reshape(
                        _H * _H, 4 * _FEAT))
            patches = jnp.concatenate(cols, axis=1)
            out = out + jnp.dot(
                patches, vt_ref[0, a * 2 + b].reshape(16 * _FEAT, 48),
                preferred_element_type=jnp.float32)
    o_ref[0] = out.reshape(_H, _H, 48)


def _v2_index():
    idx = [[[9] * 4 for _ in range(4)] for _ in range(4)]
    for a in (0, 1):
        for b in (0, 1):
            for oy in (-1, 0, 1):
                ap = (a + oy) % 2
                cy = (a + oy - ap) // 2
                iy = cy + 1 - a
                for ox in (-1, 0, 1):
                    bp = (b + ox) % 2
                    cx = (b + ox - bp) // 2
                    ix = cx + 1 - b
                    idx[a * 2 + b][iy * 2 + ix][2 * ap + bp] = \
                        (oy + 1) * 3 + (ox + 1)
    return idx


def _assemble_v2(u2r):
    idx = _v2_index()
    zero = jnp.zeros((_NSUB, _FEAT, 4 * _FEAT), jnp.float32)
    mats = []
    for ab in range(4):
        for s in range(4):
            ks = [u2r[:, int(idx[ab][s][kb])] if int(idx[ab][s][kb]) < 9
                  else zero for kb in range(4)]
            mats.append(jnp.concatenate(ks, axis=1))
    v2 = jnp.stack(mats, axis=1)
    return v2.reshape(_NSUB, 4, 4, 4 * _FEAT, 4 * _FEAT)


def _vt_index():
    idx = [[[[9] * 16 for _ in range(4)] for _ in range(4)] for _ in range(4)]
    for a in (0, 1):
        for b in (0, 1):
            for pr in range(4):
                for oy in (-1, 0, 1):
                    qr = pr + oy
                    cy = qr // 4
                    qm = qr % 4
                    if qm // 2 != a:
                        continue
                    alpha = qm % 2
                    iy = cy + a
                    for pc in range(4):
                        for ox in (-1, 0, 1):
                            qc = pc + ox
                            cx = qc // 4
                            qn = qc % 4
                            if qn // 2 != b:
                                continue
                            beta = qn % 2
                            ix = cx + b
                            idx[a * 2 + b][iy * 2 + ix][2 * alpha + beta][
                                4 * pr + pc] = (oy + 1) * 3 + (ox + 1)
    return idx


def _assemble_vt(twr):
    idx = _vt_index()
    zero = jnp.zeros((_NSUB, _FEAT, 3), jnp.float32)
    mats = []
    for ab in range(4):
        for s in range(4):
            ks = []
            for kb in range(4):
                ns = [twr[:, int(idx[ab][s][kb][p])]
                      if int(idx[ab][s][kb][p]) < 9 else zero
                      for p in range(16)]
                ks.append(jnp.concatenate(ns, axis=2))
            mats.append(jnp.concatenate(ks, axis=1))
    vt = jnp.stack(mats, axis=1)
    return vt.reshape(_NSUB, 4, 4, 4 * _FEAT, 48)


def kernel(inputs, W_feat, centroids, head_w, res1_w, res2_w, up1_w, up2_w,
           tail_w):
    n = inputs.shape[0]
    xflat = inputs.reshape(n, -1)
    route = pl.pallas_call(
        _labels_body,
        out_shape=jax.ShapeDtypeStruct((2, n), jnp.int32),
    )(xflat, W_feat, centroids.T)
    perm = route[0]
    slab = route[1]

    x = jnp.transpose(inputs, (0, 2, 3, 1))
    hw = head_w.reshape(_NSUB, 9, 3, _FEAT)
    r1 = res1_w.reshape(_NSUB, 9, _FEAT, _FEAT)
    r2 = res2_w.reshape(_NSUB, 9, _FEAT, _FEAT)
    u1 = up1_w.reshape(_NSUB, 9, _FEAT, _FEAT * 4)
    v2 = _assemble_v2(up2_w.reshape(_NSUB, 9, _FEAT, _FEAT * 4))
    vt = _assemble_vt(tail_w.reshape(_NSUB, 9, _FEAT, 3))

    def wspec(shape):
        return pl.BlockSpec(
            (1,) + shape,
            lambda i, pm, sl: (sl[i],) + (0,) * len(shape))

    out = pl.pallas_call(
        _expert_body,
        grid_spec=pltpu.PrefetchScalarGridSpec(
            num_scalar_prefetch=2,
            grid=(n,),
            in_specs=[
                pl.BlockSpec((1, _H, _H, 3), lambda i, pm, sl: (pm[i], 0, 0, 0)),
                wspec((9, 3, _FEAT)),
                wspec((9, _FEAT, _FEAT)),
                wspec((9, _FEAT, _FEAT)),
                wspec((9, _FEAT, _FEAT * 4)),
                wspec((4, 4, 4 * _FEAT, 4 * _FEAT)),
                wspec((4, 4, 4 * _FEAT, 48)),
            ],
            out_specs=pl.BlockSpec((1, _H, _H, 48),
                                   lambda i, pm, sl: (pm[i], 0, 0, 0)),
        ),
        out_shape=jax.ShapeDtypeStruct((n, _H, _H, 48), jnp.float32),
        compiler_params=pltpu.CompilerParams(
            dimension_semantics=("arbitrary",),
            vmem_limit_bytes=100 * 1024 * 1024,
        ),
    )(perm, slab, x, hw, r1, r2, u1, v2, vt)
    fine = out.reshape(n, _H, _H, 4, 4, 3)
    fine = jnp.transpose(fine, (0, 5, 1, 3, 2, 4))
    return fine.reshape(n, 3, 4 * _H, 4 * _H)
